# baseline (speedup 1.0000x reference)
import functools

import jax
import jax.numpy as jnp
from jax import lax
from jax.experimental import pallas as pl
from jax.experimental.pallas import tpu as pltpu
from jax.experimental.pallas import tpu_sc as plsc

B = 1024
C = 100000
L = 200
LANES = 16
NC = 2
NS = 16
NW = NC * NS
RPT = B // NW
NCH = 4
CH = 25088
NG = 13
LPAD = NG * LANES

_CHUNK_W = [min(CH, C - ci * CH) for ci in range(NCH)]

_mesh = plsc.VectorSubcoreMesh(core_axis_name="c", subcore_axis_name="s")


@functools.partial(
    pl.kernel,
    mesh=_mesh,
    out_type=jax.ShapeDtypeStruct((B, C), jnp.float32),
    compiler_params=pltpu.CompilerParams(
        needs_layout_passes=False, use_tc_tiling_on_sc=False
    ),
    scratch_types=(
        [pltpu.VMEM((CH,), jnp.float32) for _ in range(NCH)]
        + [pltpu.VMEM((LPAD,), jnp.int32) for _ in range(2)]
        + [pltpu.SemaphoreType.DMA for _ in range(NCH)]
    ),
)
def _multi_hot(x_hbm, out_hbm, b0, b1, b2, b3, idx0, idx1, s0, s1, s2, s3):
    bufs = [b0, b1, b2, b3]
    idxs = [idx0, idx1]
    sems = [s0, s1, s2, s3]

    wid = lax.axis_index("s") * NC + lax.axis_index("c")
    row0 = wid * RPT

    zeros16f = jnp.zeros((LANES,), jnp.float32)
    ones16f = jnp.ones((LANES,), jnp.float32)
    sent16 = jnp.full((LANES,), -1, jnp.int32)

    def zbody(i, carry):
        for cib in range(NCH):
            bufs[cib][pl.ds(i * LANES, LANES)] = zeros16f
        return carry

    lax.fori_loop(0, CH // LANES, zbody, 0)
    idx0[pl.ds(LPAD - LANES, LANES)] = sent16
    idx1[pl.ds(LPAD - LANES, LANES)] = sent16

    def add_pass(buf, idx_ref, c0):
        for g in range(NG):
            iv = idx_ref[pl.ds(g * LANES, LANES)]
            m = (iv >= c0) & (iv < c0 + CH)
            plsc.addupdate_scatter(buf, [iv - c0], ones16f, mask=m)

    def clear_pass(buf, idx_ref, c0):
        for g in range(NG):
            iv = idx_ref[pl.ds(g * LANES, LANES)]
            m = (iv >= c0) & (iv < c0 + CH)
            lv = jnp.clip(iv - c0, 0, CH - 1)
            plsc.store_scatter(buf, [lv], zeros16f, mask=m)

    def row_step(lr, parity):
        row = row0 + lr
        cur = idxs[parity]
        old = idxs[1 - parity]
        pltpu.sync_copy(x_hbm.at[row, pl.ds(0, L)], cur.at[pl.ds(0, L)])
        for ci in range(NCH):
            c0 = ci * CH
            w = _CHUNK_W[ci]

            @pl.when(lr > 0)
            def _recycle():
                pltpu.make_async_copy(
                    bufs[ci].at[pl.ds(0, w)],
                    out_hbm.at[row, pl.ds(c0, w)],
                    sems[ci],
                ).wait()
                clear_pass(bufs[ci], old, c0)

            add_pass(bufs[ci], cur, c0)
            pltpu.async_copy(
                bufs[ci].at[pl.ds(0, w)],
                out_hbm.at[row, pl.ds(c0, w)],
                sems[ci],
            )

    def body(lr2, carry):
        row_step(lr2 * 2, 0)
        row_step(lr2 * 2 + 1, 1)
        return carry

    lax.fori_loop(0, RPT // 2, body, 0)

    last = row0 + RPT - 1
    for ci in range(NCH):
        c0 = ci * CH
        w = _CHUNK_W[ci]
        pltpu.make_async_copy(
            bufs[ci].at[pl.ds(0, w)],
            out_hbm.at[last, pl.ds(c0, w)],
            sems[ci],
        ).wait()


def kernel(x):
    return _multi_hot(x)


# speedup vs baseline: 1.6829x; 1.6829x over previous
import functools

import jax
import jax.numpy as jnp
from jax import lax
from jax.experimental import pallas as pl
from jax.experimental.pallas import tpu as pltpu
from jax.experimental.pallas import tpu_sc as plsc

B = 1024
C = 100000
L = 200
LP = 256
LANES = 16
NC = 2
NS = 16
NW = NC * NS
GPT = B // (8 * NW)
CW = 4096
W24 = C - 24 * CW

_mesh = plsc.VectorSubcoreMesh(core_axis_name="c", subcore_axis_name="s")


@functools.partial(
    pl.kernel,
    mesh=_mesh,
    out_type=jax.ShapeDtypeStruct((B, C), jnp.float32),
    compiler_params=pltpu.CompilerParams(needs_layout_passes=False),
    scratch_types=(
        [pltpu.VMEM((8, CW), jnp.float32) for _ in range(2)]
        + [pltpu.VMEM((8, W24), jnp.float32)]
        + [pltpu.VMEM((8, LP), jnp.int32)]
        + [pltpu.SemaphoreType.DMA for _ in range(3)]
    ),
)
def _multi_hot(xp_hbm, out_hbm, b0, b1, b24, idx_v, s0, s1, s24):
    bufs = [b0, b1]
    sems = [s0, s1]

    wid = lax.axis_index("s") * NC + lax.axis_index("c")

    zeros16f = jnp.zeros((LANES,), jnp.float32)
    ones16f = jnp.ones((LANES,), jnp.float32)
    zeros16i = jnp.zeros((LANES,), jnp.int32)

    def zmain(i, carry):
        for b in bufs:
            for r in range(8):
                b[r, pl.ds(i * LANES, LANES)] = zeros16f
        return carry

    lax.fori_loop(0, CW // LANES, zmain, 0)

    def z24(i, carry):
        for r in range(8):
            b24[r, pl.ds(i * LANES, LANES)] = zeros16f
        return carry

    lax.fori_loop(0, W24 // LANES, z24, 0)

    def scatter_pass(buf, ck, add):
        lo = ck * CW
        hi = lo + CW

        def rbody(r, carry):
            r_v = zeros16i + r
            for k in range(LP // LANES):
                iv = idx_v[r, pl.ds(k * LANES, LANES)]
                m = (iv >= lo) & (iv < hi)
                i1 = iv - lo
                if add:
                    plsc.addupdate_scatter(buf, [r_v, i1], ones16f, mask=m)
                else:
                    plsc.store_scatter(buf, [r_v, i1], zeros16f, mask=m)
            return carry

        lax.fori_loop(0, 8, rbody, 0)

    for gi in range(GPT):
        grp = wid * GPT + gi
        r0 = grp * 8
        pltpu.sync_copy(xp_hbm.at[pl.ds(r0, 8), pl.ds(0, LP)], idx_v)

        def ck_body(c2, carry, r0=r0):
            for par in range(2):
                ck = c2 * 2 + par
                buf, sem = bufs[par], sems[par]

                @pl.when(c2 > 0)
                def _recycle():
                    pltpu.make_async_copy(
                        buf,
                        out_hbm.at[pl.ds(r0, 8), pl.ds((ck - 2) * CW, CW)],
                        sem,
                    ).wait()
                    scatter_pass(buf, ck - 2, add=False)

                scatter_pass(buf, ck, add=True)
                pltpu.async_copy(
                    buf,
                    out_hbm.at[pl.ds(r0, 8), pl.ds(ck * CW, CW)],
                    sem,
                )
            return carry

        lax.fori_loop(0, 12, ck_body, 0)

        scatter_pass(b24, 24, add=True)
        pltpu.async_copy(
            b24, out_hbm.at[pl.ds(r0, 8), pl.ds(24 * CW, W24)], s24
        )

        pltpu.make_async_copy(
            bufs[0], out_hbm.at[pl.ds(r0, 8), pl.ds(22 * CW, CW)], sems[0]
        ).wait()
        scatter_pass(bufs[0], 22, add=False)
        pltpu.make_async_copy(
            bufs[1], out_hbm.at[pl.ds(r0, 8), pl.ds(23 * CW, CW)], sems[1]
        ).wait()
        scatter_pass(bufs[1], 23, add=False)
        pltpu.make_async_copy(
            b24, out_hbm.at[pl.ds(r0, 8), pl.ds(24 * CW, W24)], s24
        ).wait()
        scatter_pass(b24, 24, add=False)


def kernel(x):
    xp = jnp.pad(x, ((0, 0), (0, LP - L)), constant_values=-1)
    return _multi_hot(xp)


# speedup vs baseline: 1.9343x; 1.1494x over previous
import functools

import jax
import jax.numpy as jnp
from jax import lax
from jax.experimental import pallas as pl
from jax.experimental.pallas import tpu as pltpu
from jax.experimental.pallas import tpu_sc as plsc

B = 1024
C = 100000
L = 200
LP = 256
LANES = 16
NC = 2
NS = 16
NW = NC * NS
GPT = B // (8 * NW)
CW = 7680
W24 = C - 13 * CW

_mesh = plsc.VectorSubcoreMesh(core_axis_name="c", subcore_axis_name="s")


@functools.partial(
    pl.kernel,
    mesh=_mesh,
    out_type=jax.ShapeDtypeStruct((B, C), jnp.float32),
    compiler_params=pltpu.CompilerParams(needs_layout_passes=False),
    scratch_types=(
        [pltpu.VMEM((8, CW), jnp.float32) for _ in range(2)]
        + [pltpu.VMEM((8, W24), jnp.float32)]
        + [pltpu.VMEM((8, LP), jnp.int32)]
        + [pltpu.SemaphoreType.DMA for _ in range(3)]
    ),
)
def _multi_hot(xp_hbm, out_hbm, b0, b1, b24, idx_v, s0, s1, s24):
    bufs = [b0, b1]
    sems = [s0, s1]

    wid = lax.axis_index("s") * NC + lax.axis_index("c")

    zeros16f = jnp.zeros((LANES,), jnp.float32)
    ones16f = jnp.ones((LANES,), jnp.float32)
    zeros16i = jnp.zeros((LANES,), jnp.int32)

    def zmain(i, carry):
        for b in bufs:
            for r in range(8):
                b[r, pl.ds(i * LANES, LANES)] = zeros16f
        return carry

    lax.fori_loop(0, CW // LANES, zmain, 0)

    def z24(i, carry):
        for r in range(8):
            b24[r, pl.ds(i * LANES, LANES)] = zeros16f
        return carry

    lax.fori_loop(0, W24 // LANES, z24, 0)

    def scatter_pass(buf, ck, add):
        lo = ck * CW
        hi = lo + CW

        def rbody(r, carry):
            r_v = zeros16i + r
            for k in range(LP // LANES):
                iv = idx_v[r, pl.ds(k * LANES, LANES)]
                m = (iv >= lo) & (iv < hi)
                i1 = iv - lo
                if add:
                    plsc.addupdate_scatter(buf, [r_v, i1], ones16f, mask=m)
                else:
                    plsc.store_scatter(buf, [r_v, i1], zeros16f, mask=m)
            return carry

        lax.fori_loop(0, 8, rbody, 0)

    for gi in range(GPT):
        grp = wid * GPT + gi
        r0 = grp * 8
        pltpu.sync_copy(xp_hbm.at[pl.ds(r0, 8), pl.ds(0, LP)], idx_v)

        def ck_body(c2, carry, r0=r0):
            for par in range(2):
                ck = c2 * 2 + par
                buf, sem = bufs[par], sems[par]

                @pl.when(c2 > 0)
                def _recycle():
                    pltpu.make_async_copy(
                        buf,
                        out_hbm.at[pl.ds(r0, 8), pl.ds((ck - 2) * CW, CW)],
                        sem,
                    ).wait()
                    scatter_pass(buf, ck - 2, add=False)

                scatter_pass(buf, ck, add=True)
                pltpu.async_copy(
                    buf,
                    out_hbm.at[pl.ds(r0, 8), pl.ds(ck * CW, CW)],
                    sem,
                )
            return carry

        lax.fori_loop(0, 6, ck_body, 0)

        pltpu.make_async_copy(
            bufs[0], out_hbm.at[pl.ds(r0, 8), pl.ds(10 * CW, CW)], sems[0]
        ).wait()
        scatter_pass(bufs[0], 10, add=False)
        scatter_pass(bufs[0], 12, add=True)
        pltpu.async_copy(
            bufs[0], out_hbm.at[pl.ds(r0, 8), pl.ds(12 * CW, CW)], sems[0]
        )

        scatter_pass(b24, 13, add=True)
        pltpu.async_copy(
            b24, out_hbm.at[pl.ds(r0, 8), pl.ds(13 * CW, W24)], s24
        )

        pltpu.make_async_copy(
            bufs[1], out_hbm.at[pl.ds(r0, 8), pl.ds(11 * CW, CW)], sems[1]
        ).wait()
        scatter_pass(bufs[1], 11, add=False)
        pltpu.make_async_copy(
            bufs[0], out_hbm.at[pl.ds(r0, 8), pl.ds(12 * CW, CW)], sems[0]
        ).wait()
        scatter_pass(bufs[0], 12, add=False)
        pltpu.make_async_copy(
            b24, out_hbm.at[pl.ds(r0, 8), pl.ds(13 * CW, W24)], s24
        ).wait()
        scatter_pass(b24, 13, add=False)


def kernel(x):
    xp = jnp.pad(x, ((0, 0), (0, LP - L)), constant_values=-1)
    return _multi_hot(xp)


# speedup vs baseline: 1.9744x; 1.0207x over previous
import functools

import jax
import jax.numpy as jnp
from jax import lax
from jax.experimental import pallas as pl
from jax.experimental.pallas import tpu as pltpu
from jax.experimental.pallas import tpu_sc as plsc

B = 1024
C = 100000
L = 200
LP = 256
LANES = 16
NC = 2
NS = 16
NW = NC * NS
GPT = B // (8 * NW)
CW = 7680
W13 = C - 13 * CW

_mesh = plsc.VectorSubcoreMesh(core_axis_name="c", subcore_axis_name="s")


@functools.partial(
    pl.kernel,
    mesh=_mesh,
    out_type=jax.ShapeDtypeStruct((B, C), jnp.float32),
    compiler_params=pltpu.CompilerParams(needs_layout_passes=False),
    scratch_types=(
        [pltpu.VMEM((8, CW), jnp.float32) for _ in range(2)]
        + [pltpu.VMEM((8, W13), jnp.float32)]
        + [pltpu.VMEM((8, LP), jnp.int32) for _ in range(2)]
        + [pltpu.SemaphoreType.DMA for _ in range(3)]
    ),
)
def _multi_hot(xp_hbm, out_hbm, b0, b1, b13, idx0, idx1, s0, s1, s13):
    bufs = [b0, b1]
    sems = [s0, s1]
    idxs = [idx0, idx1]

    wid = lax.axis_index("s") * NC + lax.axis_index("c")

    zeros16f = jnp.zeros((LANES,), jnp.float32)
    ones16f = jnp.ones((LANES,), jnp.float32)
    zeros16i = jnp.zeros((LANES,), jnp.int32)

    def zmain(i, carry):
        for b in bufs:
            for r in range(8):
                b[r, pl.ds(i * LANES, LANES)] = zeros16f
        return carry

    lax.fori_loop(0, CW // LANES, zmain, 0)

    def z13(i, carry):
        for r in range(8):
            b13[r, pl.ds(i * LANES, LANES)] = zeros16f
        return carry

    lax.fori_loop(0, W13 // LANES, z13, 0)

    def scatter_pass(buf, idx_v, ck, add):
        lo = ck * CW
        hi = lo + CW

        def rbody(r, carry):
            r_v = zeros16i + r
            for k in range(LP // LANES):
                iv = idx_v[r, pl.ds(k * LANES, LANES)]
                m = (iv >= lo) & (iv < hi)
                i1 = iv - lo
                if add:
                    plsc.addupdate_scatter(buf, [r_v, i1], ones16f, mask=m)
                else:
                    plsc.store_scatter(buf, [r_v, i1], zeros16f, mask=m)
            return carry

        lax.fori_loop(0, 8, rbody, 0)

    last_r0 = (wid * GPT + GPT - 1) * 8
    for gi in range(GPT):
        grp = wid * GPT + gi
        r0 = grp * 8
        cur = idxs[gi % 2]
        old = idxs[1 - gi % 2]
        pltpu.sync_copy(xp_hbm.at[pl.ds(r0, 8), pl.ds(0, LP)], cur)

        def dst(ck, w, r0=r0):
            return out_hbm.at[pl.ds(r0, 8), pl.ds(ck * CW, w)]

        def ck_body(c2, carry, dst=dst, cur=cur, old=old, gi=gi):
            for par in range(2):
                ck = c2 * 2 + par
                buf, sem = bufs[par], sems[par]

                if gi > 0:
                    pck = 12 if par == 0 else 11

                    @pl.when(c2 == 0)
                    def _recycle_prev():
                        pltpu.make_async_copy(buf, dst(pck, CW), sem).wait()
                        scatter_pass(buf, old, pck, add=False)

                @pl.when(c2 > 0)
                def _recycle():
                    pltpu.make_async_copy(buf, dst(ck - 2, CW), sem).wait()
                    scatter_pass(buf, cur, ck - 2, add=False)

                scatter_pass(buf, cur, ck, add=True)
                pltpu.async_copy(buf, dst(ck, CW), sem)
            return carry

        lax.fori_loop(0, 6, ck_body, 0)

        pltpu.make_async_copy(bufs[0], dst(10, CW), sems[0]).wait()
        scatter_pass(bufs[0], cur, 10, add=False)
        scatter_pass(bufs[0], cur, 12, add=True)
        pltpu.async_copy(bufs[0], dst(12, CW), sems[0])

        if gi > 0:
            pltpu.make_async_copy(b13, dst(13, W13), s13).wait()
            scatter_pass(b13, old, 13, add=False)
        scatter_pass(b13, cur, 13, add=True)
        pltpu.async_copy(b13, dst(13, W13), s13)

    def ldst(ck, w):
        return out_hbm.at[pl.ds(last_r0, 8), pl.ds(ck * CW, w)]

    pltpu.make_async_copy(bufs[1], ldst(11, CW), sems[1]).wait()
    pltpu.make_async_copy(bufs[0], ldst(12, CW), sems[0]).wait()
    pltpu.make_async_copy(b13, ldst(13, W13), s13).wait()


def kernel(x):
    xp = jnp.pad(x, ((0, 0), (0, LP - L)), constant_values=-1)
    return _multi_hot(xp)


# speedup vs baseline: 1.9762x; 1.0009x over previous
import functools

import jax
import jax.numpy as jnp
from jax import lax
from jax.experimental import pallas as pl
from jax.experimental.pallas import tpu as pltpu
from jax.experimental.pallas import tpu_sc as plsc

B = 1024
C = 100000
L = 200
LP = 256
LANES = 16
NC = 2
NS = 16
NW = NC * NS
GPT = B // (8 * NW)
CW = 7680
W13 = C - 13 * CW

_mesh = plsc.VectorSubcoreMesh(core_axis_name="c", subcore_axis_name="s")


@functools.partial(
    pl.kernel,
    mesh=_mesh,
    out_type=jax.ShapeDtypeStruct((B, C), jnp.float32),
    compiler_params=pltpu.CompilerParams(needs_layout_passes=False),
    scratch_types=(
        [pltpu.VMEM((8, CW), jnp.float32) for _ in range(2)]
        + [pltpu.VMEM((8, W13), jnp.float32)]
        + [pltpu.VMEM((8, LP), jnp.int32) for _ in range(2)]
        + [pltpu.SemaphoreType.DMA for _ in range(4)]
    ),
)
def _multi_hot(xp_hbm, out_hbm, b0, b1, b13, idx0, idx1, s0, s1, s13, spf):
    bufs = [b0, b1]
    sems = [s0, s1]
    idxs = [idx0, idx1]

    wid = lax.axis_index("s") * NC + lax.axis_index("c")

    zeros16f = jnp.zeros((LANES,), jnp.float32)
    ones16f = jnp.ones((LANES,), jnp.float32)
    zeros16i = jnp.zeros((LANES,), jnp.int32)

    def zmain(i, carry):
        for b in bufs:
            for r in range(8):
                b[r, pl.ds(i * LANES, LANES)] = zeros16f
        return carry

    lax.fori_loop(0, CW // LANES, zmain, 0)

    def z13(i, carry):
        for r in range(8):
            b13[r, pl.ds(i * LANES, LANES)] = zeros16f
        return carry

    lax.fori_loop(0, W13 // LANES, z13, 0)

    def scatter_pass(buf, idx_v, ck, add):
        lo = ck * CW
        hi = lo + CW

        def rbody(r, carry):
            r_v = zeros16i + r
            for k in range(LP // LANES):
                iv = idx_v[r, pl.ds(k * LANES, LANES)]
                m = (iv >= lo) & (iv < hi)
                i1 = iv - lo
                if add:
                    plsc.addupdate_scatter(buf, [r_v, i1], ones16f, mask=m)
                else:
                    plsc.store_scatter(buf, [r_v, i1], zeros16f, mask=m)
            return carry

        lax.fori_loop(0, 8, rbody, 0)

    def idx_src(g):
        return xp_hbm.at[pl.ds(g * 8, 8), pl.ds(0, LP)]

    last_r0 = (wid * GPT + GPT - 1) * 8
    pltpu.async_copy(idx_src(wid * GPT), idxs[0], spf)
    for gi in range(GPT):
        grp = wid * GPT + gi
        r0 = grp * 8
        cur = idxs[gi % 2]
        old = idxs[1 - gi % 2]
        pltpu.make_async_copy(idx_src(grp), cur, spf).wait()

        def dst(ck, w, r0=r0):
            return out_hbm.at[pl.ds(r0, 8), pl.ds(ck * CW, w)]

        def ck_body(c2, carry, dst=dst, cur=cur, old=old, gi=gi):
            for par in range(2):
                ck = c2 * 2 + par
                buf, sem = bufs[par], sems[par]

                if gi > 0:
                    pck = 12 if par == 0 else 11

                    @pl.when(c2 == 0)
                    def _recycle_prev():
                        pltpu.make_async_copy(buf, dst(pck, CW), sem).wait()
                        scatter_pass(buf, old, pck, add=False)

                @pl.when(c2 > 0)
                def _recycle():
                    pltpu.make_async_copy(buf, dst(ck - 2, CW), sem).wait()
                    scatter_pass(buf, cur, ck - 2, add=False)

                scatter_pass(buf, cur, ck, add=True)
                pltpu.async_copy(buf, dst(ck, CW), sem)
            return carry

        lax.fori_loop(0, 6, ck_body, 0)

        pltpu.make_async_copy(bufs[0], dst(10, CW), sems[0]).wait()
        scatter_pass(bufs[0], cur, 10, add=False)
        scatter_pass(bufs[0], cur, 12, add=True)
        pltpu.async_copy(bufs[0], dst(12, CW), sems[0])

        if gi > 0:
            pltpu.make_async_copy(b13, dst(13, W13), s13).wait()
            scatter_pass(b13, old, 13, add=False)
        if gi + 1 < GPT:
            pltpu.async_copy(idx_src(grp + 1), old, spf)
        scatter_pass(b13, cur, 13, add=True)
        pltpu.async_copy(b13, dst(13, W13), s13)

    def ldst(ck, w):
        return out_hbm.at[pl.ds(last_r0, 8), pl.ds(ck * CW, w)]

    pltpu.make_async_copy(bufs[1], ldst(11, CW), sems[1]).wait()
    pltpu.make_async_copy(bufs[0], ldst(12, CW), sems[0]).wait()
    pltpu.make_async_copy(b13, ldst(13, W13), s13).wait()


def kernel(x):
    xp = jnp.pad(x, ((0, 0), (0, LP - L)), constant_values=-1)
    return _multi_hot(xp)
